# Initial kernel scaffold; baseline (speedup 1.0000x reference)
import jax
import jax.numpy as jnp
from jax.experimental import pallas as pl


def kernel(h, edge_index, t_span, W1, b1, Wa, ba, Wb, bb, W2, b2):
    raise NotImplementedError("write your pallas kernel here")


# speedup vs baseline: 2.5083x; 2.5083x over previous
import functools

import jax
import jax.numpy as jnp
from jax import lax
from jax.experimental import pallas as pl
from jax.experimental.pallas import tpu as pltpu
from jax.experimental.pallas import tpu_sc as plsc

N = 10000
NP = 10240
E = 320000
EP = 327680
EBLK = 128
NBLK = EP // EBLK
PAD_NODE = 10200
RB = 256
GRID = NP // RB
HIGH = jax.lax.Precision.HIGHEST

_mesh = plsc.VectorSubcoreMesh(core_axis_name="c", subcore_axis_name="s")


def _make_hist():
    per_tile = NBLK // 16

    @functools.partial(
        pl.kernel,
        out_type=jax.ShapeDtypeStruct((2 * NP, 128), jnp.float32),
        mesh=_mesh,
        scratch_types=[
            pltpu.VMEM((1, EBLK), jnp.int32),
            pltpu.VMEM((EBLK, 128), jnp.float32),
            pltpu.VMEM((EBLK, 128), jnp.float32),
            pltpu.VMEM_SHARED((NP, 128), jnp.float32),
        ],
    )
    def hist(hidx_hbm, ones_hbm, zeros_hbm, cnt_hbm, idx, ones_v, tmp, acc):
        c = lax.axis_index("c")
        s = lax.axis_index("s")
        pltpu.sync_copy(ones_hbm, ones_v)
        pltpu.sync_copy(zeros_hbm, tmp)
        rows_per_tile = NP // 16

        @pl.loop(0, rows_per_tile // EBLK)
        def _(j):
            pltpu.sync_copy(tmp, acc.at[pl.ds(s * rows_per_tile + j * EBLK, EBLK)])

        plsc.subcore_barrier()

        @pl.loop(0, per_tile)
        def _(b):
            e0 = (s * per_tile + b) * EBLK
            pltpu.sync_copy(hidx_hbm.at[pl.ds(c * EP + e0, EBLK)], idx.at[0])
            pltpu.sync_copy(ones_v, acc.at[idx.at[0]], add=True)

        plsc.subcore_barrier()

        @pl.loop(0, rows_per_tile // EBLK)
        def _(j):
            base = s * rows_per_tile + j * EBLK
            pltpu.sync_copy(acc.at[pl.ds(base, EBLK)], tmp)
            pltpu.sync_copy(tmp, cnt_hbm.at[pl.ds(c * NP + base, EBLK)])

    return hist


def _make_spmm(hw):
    per_tile = NBLK // 16
    rows_per_tile = NP // 16

    @functools.partial(
        pl.kernel,
        out_type=jax.ShapeDtypeStruct((2 * NP, hw), jnp.float32),
        mesh=_mesh,
        scratch_types=[
            pltpu.VMEM((1, EBLK), jnp.int32),
            pltpu.VMEM((1, EBLK), jnp.int32),
            pltpu.VMEM((EBLK, hw), jnp.float32),
            pltpu.VMEM((EBLK, hw), jnp.float32),
            pltpu.VMEM_SHARED((NP, hw), jnp.float32),
        ],
    )
    def spmm(y_hbm, src2_hbm, dst_hbm, zeros_hbm, g_hbm, sidx, didx, rows, tmp, acc):
        c = lax.axis_index("c")
        s = lax.axis_index("s")
        pltpu.sync_copy(zeros_hbm, tmp)

        @pl.loop(0, rows_per_tile // EBLK)
        def _(j):
            pltpu.sync_copy(tmp, acc.at[pl.ds(s * rows_per_tile + j * EBLK, EBLK)])

        plsc.subcore_barrier()

        @pl.loop(0, per_tile)
        def _(b):
            e0 = (s * per_tile + b) * EBLK
            pltpu.sync_copy(src2_hbm.at[pl.ds(c * EP + e0, EBLK)], sidx.at[0])
            pltpu.sync_copy(dst_hbm.at[pl.ds(e0, EBLK)], didx.at[0])
            pltpu.sync_copy(y_hbm.at[sidx.at[0]], rows)
            pltpu.sync_copy(rows, acc.at[didx.at[0]], add=True)

        plsc.subcore_barrier()

        @pl.loop(0, rows_per_tile // EBLK)
        def _(j):
            base = s * rows_per_tile + j * EBLK
            pltpu.sync_copy(acc.at[pl.ds(base, EBLK)], tmp)
            pltpu.sync_copy(tmp, g_hbm.at[pl.ds(c * NP + base, EBLK)])

    return spmm


def _make_spmm_edgesplit():
    per_tile = NBLK // 32
    rows_per_tile = NP // 16

    @functools.partial(
        pl.kernel,
        out_type=jax.ShapeDtypeStruct((2 * NP, 128), jnp.float32),
        mesh=_mesh,
        scratch_types=[
            pltpu.VMEM((1, EBLK), jnp.int32),
            pltpu.VMEM((1, EBLK), jnp.int32),
            pltpu.VMEM((EBLK, 128), jnp.float32),
            pltpu.VMEM((EBLK, 128), jnp.float32),
            pltpu.VMEM_SHARED((NP, 128), jnp.float32),
        ],
    )
    def spmm(y_hbm, src_hbm, dst_hbm, zeros_hbm, g_hbm, sidx, didx, rows, tmp, acc):
        c = lax.axis_index("c")
        s = lax.axis_index("s")
        pltpu.sync_copy(zeros_hbm, tmp)

        @pl.loop(0, rows_per_tile // EBLK)
        def _(j):
            pltpu.sync_copy(tmp, acc.at[pl.ds(s * rows_per_tile + j * EBLK, EBLK)])

        plsc.subcore_barrier()

        @pl.loop(0, per_tile)
        def _(b):
            e0 = ((c * 16 + s) * per_tile + b) * EBLK
            pltpu.sync_copy(src_hbm.at[pl.ds(e0, EBLK)], sidx.at[0])
            pltpu.sync_copy(dst_hbm.at[pl.ds(e0, EBLK)], didx.at[0])
            pltpu.sync_copy(y_hbm.at[sidx.at[0]], rows)
            pltpu.sync_copy(rows, acc.at[didx.at[0]], add=True)

        plsc.subcore_barrier()

        @pl.loop(0, rows_per_tile // EBLK)
        def _(j):
            base = s * rows_per_tile + j * EBLK
            pltpu.sync_copy(acc.at[pl.ds(base, EBLK)], tmp)
            pltpu.sync_copy(tmp, g_hbm.at[pl.ds(c * NP + base, EBLK)])

    return spmm


_hist = _make_hist()
_spmm128 = _make_spmm(128)
_spmm_es = _make_spmm_edgesplit()


def _softplus(v):
    return jnp.maximum(v, 0.0) + jnp.log1p(jnp.exp(-jnp.abs(v)))


def _dot(x, w):
    return jnp.dot(x, w, precision=HIGH, preferred_element_type=jnp.float32)


def _split3(shape_hw):
    return pl.BlockSpec((2, RB, shape_hw), lambda i: (0, i, 0))


_spec_n1 = pl.BlockSpec((RB, 1), lambda i: (i, 0))


def _spec_full(shape):
    return pl.BlockSpec(shape, lambda i: tuple(0 for _ in shape))


def _cat(a, b):
    return jnp.concatenate([a, b], axis=1)


def _split_write(ref, val, hw):
    ref[0] = val[:, :hw]
    ref[1] = val[:, hw:]


def _norm_body(cnt_ref, dsrc_ref, ddst_ref):
    cnt = cnt_ref[...]
    dsrc = cnt[0:NP, 0:1]
    ddst = cnt[NP:2 * NP, 0:1]
    rows = lax.broadcasted_iota(jnp.int32, (NP, 1), 0)
    valid = rows < N
    dsrc_ref[...] = jnp.where(valid, lax.rsqrt(dsrc + 1.0), 0.0)
    ddst_ref[...] = jnp.where(valid, lax.rsqrt(ddst + 1.0), 0.0)


def _tc_norms(cnt):
    return pl.pallas_call(
        _norm_body,
        grid=(1,),
        in_specs=[_spec_full((2 * NP, 128))],
        out_specs=[pl.BlockSpec((NP, 1), lambda i: (0, 0))] * 2,
        out_shape=[jax.ShapeDtypeStruct((NP, 1), jnp.float32)] * 2,
    )(cnt)


def _in_body(x_ref, w_ref, b_ref, ds_ref, y_ref):
    y = (_dot(x_ref[...], w_ref[...]) + b_ref[...]) * ds_ref[...]
    _split_write(y_ref, y, 128)


def _tc_in(h_pad, W1, b1, dsrc):
    return pl.pallas_call(
        _in_body,
        grid=(GRID,),
        in_specs=[
            pl.BlockSpec((RB, 128), lambda i: (i, 0)),
            _spec_full((128, 256)),
            _spec_full((1, 256)),
            _spec_n1,
        ],
        out_specs=_split3(128),
        out_shape=jax.ShapeDtypeStruct((2, NP, 128), jnp.float32),
    )(h_pad, W1, b1, dsrc)


def _first_body(g_ref, y_ref, dd_ref, w_ref, b_ref, ds_ref, z_ref, ya_ref):
    dd = dd_ref[...]
    z0 = dd * (g_ref[0] + y_ref[0])
    z1 = dd * (g_ref[1] + y_ref[1])
    z_ref[0] = z0
    z_ref[1] = z1
    ya = (_dot(_cat(z0, z1), w_ref[...]) + b_ref[...]) * ds_ref[...]
    _split_write(ya_ref, ya, 128)


def _tc_first(g, y, ddst, Wa, ba, dsrc):
    return pl.pallas_call(
        _first_body,
        grid=(GRID,),
        in_specs=[
            _split3(128), _split3(128), _spec_n1,
            _spec_full((256, 256)), _spec_full((1, 256)), _spec_n1,
        ],
        out_specs=[_split3(128), _split3(128)],
        out_shape=[jax.ShapeDtypeStruct((2, NP, 128), jnp.float32)] * 2,
    )(g, y, ddst, Wa, ba, dsrc)


def _b_body(g_ref, y_ref, dd_ref, w_ref, b_ref, ds_ref, yb_ref):
    dd = dd_ref[...]
    x0 = dd * (g_ref[0] + y_ref[0])
    x1 = dd * (g_ref[1] + y_ref[1])
    x = _softplus(_cat(x0, x1))
    yb = (_dot(x, w_ref[...]) + b_ref[...]) * ds_ref[...]
    _split_write(yb_ref, yb, 128)


def _tc_b(g, y, ddst, Wb, bb, dsrc):
    return pl.pallas_call(
        _b_body,
        grid=(GRID,),
        in_specs=[
            _split3(128), _split3(128), _spec_n1,
            _spec_full((256, 256)), _spec_full((1, 256)), _spec_n1,
        ],
        out_specs=_split3(128),
        out_shape=jax.ShapeDtypeStruct((2, NP, 128), jnp.float32),
    )(g, y, ddst, Wb, bb, dsrc)


def _c_mid_body(g_ref, y_ref, z_ref, dd_ref, w_ref, b_ref, ds_ref, dtc_ref,
                k_ref, ya_ref):
    dd = dd_ref[...]
    dtc = dtc_ref[0, 0]
    k0 = dd * (g_ref[0] + y_ref[0])
    k1 = dd * (g_ref[1] + y_ref[1])
    k_ref[0] = k0
    k_ref[1] = k1
    u = _cat(z_ref[0] + dtc * k0, z_ref[1] + dtc * k1)
    ya = (_dot(u, w_ref[...]) + b_ref[...]) * ds_ref[...]
    _split_write(ya_ref, ya, 128)


def _tc_c_mid(g, y, z, ddst, Wa, ba, dsrc, dtc):
    return pl.pallas_call(
        _c_mid_body,
        grid=(GRID,),
        in_specs=[
            _split3(128), _split3(128), _split3(128), _spec_n1,
            _spec_full((256, 256)), _spec_full((1, 256)), _spec_n1,
            _spec_full((1, 1)),
        ],
        out_specs=[_split3(128), _split3(128)],
        out_shape=[jax.ShapeDtypeStruct((2, NP, 128), jnp.float32)] * 2,
    )(g, y, z, ddst, Wa, ba, dsrc, dtc)


def _c_last_body(g_ref, y_ref, z_ref, k1_ref, k2_ref, k3_ref, dd_ref,
                 w_ref, b_ref, ds_ref, dt6_ref, zn_ref, ya_ref, *, split_out):
    dd = dd_ref[...]
    dt6 = dt6_ref[0, 0]
    zn = []
    for hh in range(2):
        k4 = dd * (g_ref[hh] + y_ref[hh])
        znh = z_ref[hh] + dt6 * (k1_ref[hh] + 2.0 * k2_ref[hh]
                                 + 2.0 * k3_ref[hh] + k4)
        zn_ref[hh] = znh
        zn.append(znh)
    ya = (_dot(_cat(zn[0], zn[1]), w_ref[...]) + b_ref[...]) * ds_ref[...]
    if split_out:
        _split_write(ya_ref, ya, 128)
    else:
        ya_ref[...] = ya


def _tc_c_last(g, y, z, k1, k2, k3, ddst, W, b, dsrc, dt6, split_out):
    wout = W.shape[1]
    if split_out:
        ya_spec = _split3(128)
        ya_shape = jax.ShapeDtypeStruct((2, NP, 128), jnp.float32)
    else:
        ya_spec = pl.BlockSpec((RB, wout), lambda i: (i, 0))
        ya_shape = jax.ShapeDtypeStruct((NP, wout), jnp.float32)
    return pl.pallas_call(
        functools.partial(_c_last_body, split_out=split_out),
        grid=(GRID,),
        in_specs=[
            _split3(128), _split3(128), _split3(128), _split3(128),
            _split3(128), _split3(128), _spec_n1,
            _spec_full((256, wout)), _spec_full((1, wout)), _spec_n1,
            _spec_full((1, 1)),
        ],
        out_specs=[_split3(128), ya_spec],
        out_shape=[jax.ShapeDtypeStruct((2, NP, 128), jnp.float32), ya_shape],
    )(g, y, z, k1, k2, k3, ddst, W, b, dsrc, dt6)


def _out_body(g_ref, y_ref, dd_ref, o_ref):
    o_ref[...] = dd_ref[...] * (g_ref[0] + g_ref[1] + y_ref[...])


def _tc_out(g, y, ddst):
    return pl.pallas_call(
        _out_body,
        grid=(GRID,),
        in_specs=[_split3(128), pl.BlockSpec((RB, 128), lambda i: (i, 0)),
                  _spec_n1],
        out_specs=pl.BlockSpec((RB, 128), lambda i: (i, 0)),
        out_shape=jax.ShapeDtypeStruct((NP, 128), jnp.float32),
    )(g, y, ddst)


def kernel(h, edge_index, t_span, W1, b1, Wa, ba, Wb, bb, W2, b2):
    f32 = jnp.float32
    src = edge_index[0]
    dst = edge_index[1]
    pad = jnp.full((EP - E,), PAD_NODE, jnp.int32)
    src_p = jnp.concatenate([src, pad])
    dst_p = jnp.concatenate([dst, pad])
    src2 = jnp.concatenate([src_p, src_p + NP])
    hist_idx = jnp.concatenate([src_p, dst_p])

    ones128 = jnp.ones((EBLK, 128), f32)
    zeros128 = jnp.zeros((EBLK, 128), f32)

    cnt = _hist(hist_idx, ones128, zeros128)
    dsrc, ddst = _tc_norms(cnt)

    h_pad = jnp.concatenate([h, jnp.zeros((NP - N, h.shape[1]), f32)])
    W1r, bar, bbr = W1, ba.reshape(1, -1), bb.reshape(1, -1)
    b1r, b2r = b1.reshape(1, -1), b2.reshape(1, -1)

    def spmm128(y3):
        g = _spmm128(y3.reshape(2 * NP, 128), src2, dst_p, zeros128)
        return g.reshape(2, NP, 128)

    dts = t_span[1:] - t_span[:-1]

    y1 = _tc_in(h_pad, W1r, b1r, dsrc)
    g1 = spmm128(y1)
    z, ya = _tc_first(g1, y1, ddst, Wa, bar, dsrc)

    n_steps = t_span.shape[0] - 1
    yo = None
    for i in range(n_steps):
        dt = dts[i]
        ks = []
        for s_idx in range(4):
            ga = spmm128(ya)
            yb = _tc_b(ga, ya, ddst, Wb, bbr, dsrc)
            gb = spmm128(yb)
            if s_idx < 3:
                coeff = 0.5 if s_idx < 2 else 1.0
                dtc = (coeff * dt).reshape(1, 1)
                k, ya = _tc_c_mid(gb, yb, z, ddst, Wa, bar, dsrc, dtc)
                ks.append(k)
            else:
                dt6 = (dt / 6.0).reshape(1, 1)
                if i < n_steps - 1:
                    z, ya = _tc_c_last(gb, yb, z, ks[0], ks[1], ks[2],
                                       ddst, Wa, bar, dsrc, dt6, True)
                else:
                    z, yo = _tc_c_last(gb, yb, z, ks[0], ks[1], ks[2],
                                       ddst, W2, b2r, dsrc, dt6, False)

    go = _spmm_es(yo, src_p, dst_p, zeros128)
    go = go.reshape(2, NP, 128)
    out = _tc_out(go, yo, ddst)
    return out[:N]


# speedup vs baseline: 3.4013x; 1.3560x over previous
import functools

import jax
import jax.numpy as jnp
from jax import lax
from jax.experimental import pallas as pl
from jax.experimental.pallas import tpu as pltpu
from jax.experimental.pallas import tpu_sc as plsc

N = 10000
NP = 10240
E = 320000
EP = 327680
EBLK = 128
NBLK = EP // EBLK
PAD_NODE = 10200
RB = 256
GRID = NP // RB
HIGH = jax.lax.Precision.HIGHEST

_mesh = plsc.VectorSubcoreMesh(core_axis_name="c", subcore_axis_name="s")


def _make_hist():
    per_tile = NBLK // 16

    @functools.partial(
        pl.kernel,
        out_type=jax.ShapeDtypeStruct((2 * NP, 128), jnp.float32),
        mesh=_mesh,
        scratch_types=[
            pltpu.VMEM((1, EBLK), jnp.int32),
            pltpu.VMEM((EBLK, 128), jnp.float32),
            pltpu.VMEM((EBLK, 128), jnp.float32),
            pltpu.VMEM_SHARED((NP, 128), jnp.float32),
        ],
    )
    def hist(hidx_hbm, ones_hbm, zeros_hbm, cnt_hbm, idx, ones_v, tmp, acc):
        c = lax.axis_index("c")
        s = lax.axis_index("s")
        pltpu.sync_copy(ones_hbm, ones_v)
        pltpu.sync_copy(zeros_hbm, tmp)
        rows_per_tile = NP // 16

        @pl.loop(0, rows_per_tile // EBLK)
        def _(j):
            pltpu.sync_copy(tmp, acc.at[pl.ds(s * rows_per_tile + j * EBLK, EBLK)])

        plsc.subcore_barrier()

        @pl.loop(0, per_tile)
        def _(b):
            e0 = (s * per_tile + b) * EBLK
            pltpu.sync_copy(hidx_hbm.at[pl.ds(c * EP + e0, EBLK)], idx.at[0])
            pltpu.sync_copy(ones_v, acc.at[idx.at[0]], add=True)

        plsc.subcore_barrier()

        @pl.loop(0, rows_per_tile // EBLK)
        def _(j):
            base = s * rows_per_tile + j * EBLK
            pltpu.sync_copy(acc.at[pl.ds(base, EBLK)], tmp)
            pltpu.sync_copy(tmp, cnt_hbm.at[pl.ds(c * NP + base, EBLK)])

    return hist


def _make_spmm(hw):
    per_tile = NBLK // 16
    rows_per_tile = NP // 16
    ICH = 16
    n_chunks = per_tile // ICH

    @functools.partial(
        pl.kernel,
        out_type=jax.ShapeDtypeStruct((2 * NP, hw), jnp.float32),
        mesh=_mesh,
        scratch_types=[
            pltpu.VMEM((ICH, EBLK), jnp.int32),
            pltpu.VMEM((ICH, EBLK), jnp.int32),
            pltpu.VMEM((EBLK, hw), jnp.float32),
            pltpu.VMEM((EBLK, hw), jnp.float32),
            pltpu.VMEM_SHARED((NP, hw), jnp.float32),
            pltpu.SemaphoreType.DMA,
            pltpu.SemaphoreType.DMA,
        ],
    )
    def spmm(y_hbm, src2_hbm, dst_hbm, zeros_hbm, g_hbm,
             sidx, didx, rows0, rows1, acc, sem0, sem1):
        c = lax.axis_index("c")
        s = lax.axis_index("s")
        blk0 = s * per_tile
        pltpu.sync_copy(zeros_hbm, rows0)

        @pl.loop(0, rows_per_tile // EBLK)
        def _(j):
            pltpu.sync_copy(rows0, acc.at[pl.ds(s * rows_per_tile + j * EBLK, EBLK)])

        plsc.subcore_barrier()

        bufs = (rows0, rows1)
        sems = (sem0, sem1)

        @pl.loop(0, n_chunks)
        def _(ch):
            pltpu.sync_copy(src2_hbm.at[pl.ds(c * NBLK + blk0 + ch * ICH, ICH)],
                            sidx)
            pltpu.sync_copy(dst_hbm.at[pl.ds(blk0 + ch * ICH, ICH)], didx)
            pltpu.async_copy(y_hbm.at[sidx.at[0]], rows0, sem0)

            @pl.loop(0, ICH // 2)
            def _(p):
                for par in range(2):
                    b = 2 * p + par
                    buf, sem = bufs[par], sems[par]
                    nbuf, nsem = bufs[1 - par], sems[1 - par]
                    pltpu.make_async_copy(y_hbm.at[sidx.at[0]], buf, sem).wait()

                    @pl.when(b + 1 < ICH)
                    def _():
                        pltpu.async_copy(y_hbm.at[sidx.at[b + 1]], nbuf, nsem)

                    pltpu.sync_copy(buf, acc.at[didx.at[b]], add=True)

        plsc.subcore_barrier()

        @pl.loop(0, rows_per_tile // EBLK)
        def _(j):
            base = s * rows_per_tile + j * EBLK
            pltpu.sync_copy(acc.at[pl.ds(base, EBLK)], rows0)
            pltpu.sync_copy(rows0, g_hbm.at[pl.ds(c * NP + base, EBLK)])

    return spmm


def _make_spmm_edgesplit():
    per_tile = NBLK // 32
    rows_per_tile = NP // 16

    @functools.partial(
        pl.kernel,
        out_type=jax.ShapeDtypeStruct((2 * NP, 128), jnp.float32),
        mesh=_mesh,
        scratch_types=[
            pltpu.VMEM((1, EBLK), jnp.int32),
            pltpu.VMEM((1, EBLK), jnp.int32),
            pltpu.VMEM((EBLK, 128), jnp.float32),
            pltpu.VMEM((EBLK, 128), jnp.float32),
            pltpu.VMEM_SHARED((NP, 128), jnp.float32),
        ],
    )
    def spmm(y_hbm, src_hbm, dst_hbm, zeros_hbm, g_hbm, sidx, didx, rows, tmp, acc):
        c = lax.axis_index("c")
        s = lax.axis_index("s")
        pltpu.sync_copy(zeros_hbm, tmp)

        @pl.loop(0, rows_per_tile // EBLK)
        def _(j):
            pltpu.sync_copy(tmp, acc.at[pl.ds(s * rows_per_tile + j * EBLK, EBLK)])

        plsc.subcore_barrier()

        @pl.loop(0, per_tile)
        def _(b):
            e0 = ((c * 16 + s) * per_tile + b) * EBLK
            pltpu.sync_copy(src_hbm.at[pl.ds(e0, EBLK)], sidx.at[0])
            pltpu.sync_copy(dst_hbm.at[pl.ds(e0, EBLK)], didx.at[0])
            pltpu.sync_copy(y_hbm.at[sidx.at[0]], rows)
            pltpu.sync_copy(rows, acc.at[didx.at[0]], add=True)

        plsc.subcore_barrier()

        @pl.loop(0, rows_per_tile // EBLK)
        def _(j):
            base = s * rows_per_tile + j * EBLK
            pltpu.sync_copy(acc.at[pl.ds(base, EBLK)], tmp)
            pltpu.sync_copy(tmp, g_hbm.at[pl.ds(c * NP + base, EBLK)])

    return spmm


_hist = _make_hist()
_spmm128 = _make_spmm(128)
_spmm_es = _make_spmm_edgesplit()


def _softplus(v):
    return jnp.maximum(v, 0.0) + jnp.log1p(jnp.exp(-jnp.abs(v)))


def _dot(x, w):
    return jnp.dot(x, w, precision=HIGH, preferred_element_type=jnp.float32)


def _split3(shape_hw):
    return pl.BlockSpec((2, RB, shape_hw), lambda i: (0, i, 0))


_spec_n1 = pl.BlockSpec((RB, 1), lambda i: (i, 0))


def _spec_full(shape):
    return pl.BlockSpec(shape, lambda i: tuple(0 for _ in shape))


def _cat(a, b):
    return jnp.concatenate([a, b], axis=1)


def _split_write(ref, val, hw):
    ref[0] = val[:, :hw]
    ref[1] = val[:, hw:]


def _norm_body(cnt_ref, dsrc_ref, ddst_ref):
    cnt = cnt_ref[...]
    dsrc = cnt[0:NP, 0:1]
    ddst = cnt[NP:2 * NP, 0:1]
    rows = lax.broadcasted_iota(jnp.int32, (NP, 1), 0)
    valid = rows < N
    dsrc_ref[...] = jnp.where(valid, lax.rsqrt(dsrc + 1.0), 0.0)
    ddst_ref[...] = jnp.where(valid, lax.rsqrt(ddst + 1.0), 0.0)


def _tc_norms(cnt):
    return pl.pallas_call(
        _norm_body,
        grid=(1,),
        in_specs=[_spec_full((2 * NP, 128))],
        out_specs=[pl.BlockSpec((NP, 1), lambda i: (0, 0))] * 2,
        out_shape=[jax.ShapeDtypeStruct((NP, 1), jnp.float32)] * 2,
    )(cnt)


def _in_body(x_ref, w_ref, b_ref, ds_ref, y_ref):
    y = (_dot(x_ref[...], w_ref[...]) + b_ref[...]) * ds_ref[...]
    _split_write(y_ref, y, 128)


def _tc_in(h_pad, W1, b1, dsrc):
    return pl.pallas_call(
        _in_body,
        grid=(GRID,),
        in_specs=[
            pl.BlockSpec((RB, 128), lambda i: (i, 0)),
            _spec_full((128, 256)),
            _spec_full((1, 256)),
            _spec_n1,
        ],
        out_specs=_split3(128),
        out_shape=jax.ShapeDtypeStruct((2, NP, 128), jnp.float32),
    )(h_pad, W1, b1, dsrc)


def _first_body(g_ref, y_ref, dd_ref, w_ref, b_ref, ds_ref, z_ref, ya_ref):
    dd = dd_ref[...]
    z0 = dd * (g_ref[0] + y_ref[0])
    z1 = dd * (g_ref[1] + y_ref[1])
    z_ref[0] = z0
    z_ref[1] = z1
    ya = (_dot(_cat(z0, z1), w_ref[...]) + b_ref[...]) * ds_ref[...]
    _split_write(ya_ref, ya, 128)


def _tc_first(g, y, ddst, Wa, ba, dsrc):
    return pl.pallas_call(
        _first_body,
        grid=(GRID,),
        in_specs=[
            _split3(128), _split3(128), _spec_n1,
            _spec_full((256, 256)), _spec_full((1, 256)), _spec_n1,
        ],
        out_specs=[_split3(128), _split3(128)],
        out_shape=[jax.ShapeDtypeStruct((2, NP, 128), jnp.float32)] * 2,
    )(g, y, ddst, Wa, ba, dsrc)


def _b_body(g_ref, y_ref, dd_ref, w_ref, b_ref, ds_ref, yb_ref):
    dd = dd_ref[...]
    x0 = dd * (g_ref[0] + y_ref[0])
    x1 = dd * (g_ref[1] + y_ref[1])
    x = _softplus(_cat(x0, x1))
    yb = (_dot(x, w_ref[...]) + b_ref[...]) * ds_ref[...]
    _split_write(yb_ref, yb, 128)


def _tc_b(g, y, ddst, Wb, bb, dsrc):
    return pl.pallas_call(
        _b_body,
        grid=(GRID,),
        in_specs=[
            _split3(128), _split3(128), _spec_n1,
            _spec_full((256, 256)), _spec_full((1, 256)), _spec_n1,
        ],
        out_specs=_split3(128),
        out_shape=jax.ShapeDtypeStruct((2, NP, 128), jnp.float32),
    )(g, y, ddst, Wb, bb, dsrc)


def _c_mid_body(g_ref, y_ref, z_ref, dd_ref, w_ref, b_ref, ds_ref, dtc_ref,
                k_ref, ya_ref):
    dd = dd_ref[...]
    dtc = dtc_ref[0, 0]
    k0 = dd * (g_ref[0] + y_ref[0])
    k1 = dd * (g_ref[1] + y_ref[1])
    k_ref[0] = k0
    k_ref[1] = k1
    u = _cat(z_ref[0] + dtc * k0, z_ref[1] + dtc * k1)
    ya = (_dot(u, w_ref[...]) + b_ref[...]) * ds_ref[...]
    _split_write(ya_ref, ya, 128)


def _tc_c_mid(g, y, z, ddst, Wa, ba, dsrc, dtc):
    return pl.pallas_call(
        _c_mid_body,
        grid=(GRID,),
        in_specs=[
            _split3(128), _split3(128), _split3(128), _spec_n1,
            _spec_full((256, 256)), _spec_full((1, 256)), _spec_n1,
            _spec_full((1, 1)),
        ],
        out_specs=[_split3(128), _split3(128)],
        out_shape=[jax.ShapeDtypeStruct((2, NP, 128), jnp.float32)] * 2,
    )(g, y, z, ddst, Wa, ba, dsrc, dtc)


def _c_last_body(g_ref, y_ref, z_ref, k1_ref, k2_ref, k3_ref, dd_ref,
                 w_ref, b_ref, ds_ref, dt6_ref, zn_ref, ya_ref, *, split_out):
    dd = dd_ref[...]
    dt6 = dt6_ref[0, 0]
    zn = []
    for hh in range(2):
        k4 = dd * (g_ref[hh] + y_ref[hh])
        znh = z_ref[hh] + dt6 * (k1_ref[hh] + 2.0 * k2_ref[hh]
                                 + 2.0 * k3_ref[hh] + k4)
        zn_ref[hh] = znh
        zn.append(znh)
    ya = (_dot(_cat(zn[0], zn[1]), w_ref[...]) + b_ref[...]) * ds_ref[...]
    if split_out:
        _split_write(ya_ref, ya, 128)
    else:
        ya_ref[...] = ya


def _tc_c_last(g, y, z, k1, k2, k3, ddst, W, b, dsrc, dt6, split_out):
    wout = W.shape[1]
    if split_out:
        ya_spec = _split3(128)
        ya_shape = jax.ShapeDtypeStruct((2, NP, 128), jnp.float32)
    else:
        ya_spec = pl.BlockSpec((RB, wout), lambda i: (i, 0))
        ya_shape = jax.ShapeDtypeStruct((NP, wout), jnp.float32)
    return pl.pallas_call(
        functools.partial(_c_last_body, split_out=split_out),
        grid=(GRID,),
        in_specs=[
            _split3(128), _split3(128), _split3(128), _split3(128),
            _split3(128), _split3(128), _spec_n1,
            _spec_full((256, wout)), _spec_full((1, wout)), _spec_n1,
            _spec_full((1, 1)),
        ],
        out_specs=[_split3(128), ya_spec],
        out_shape=[jax.ShapeDtypeStruct((2, NP, 128), jnp.float32), ya_shape],
    )(g, y, z, k1, k2, k3, ddst, W, b, dsrc, dt6)


def _out_body(g_ref, y_ref, dd_ref, o_ref):
    o_ref[...] = dd_ref[...] * (g_ref[0] + g_ref[1] + y_ref[...])


def _tc_out(g, y, ddst):
    return pl.pallas_call(
        _out_body,
        grid=(GRID,),
        in_specs=[_split3(128), pl.BlockSpec((RB, 128), lambda i: (i, 0)),
                  _spec_n1],
        out_specs=pl.BlockSpec((RB, 128), lambda i: (i, 0)),
        out_shape=jax.ShapeDtypeStruct((NP, 128), jnp.float32),
    )(g, y, ddst)


def kernel(h, edge_index, t_span, W1, b1, Wa, ba, Wb, bb, W2, b2):
    f32 = jnp.float32
    src = edge_index[0]
    dst = edge_index[1]
    pad = jnp.full((EP - E,), PAD_NODE, jnp.int32)
    src_p = jnp.concatenate([src, pad])
    dst_p = jnp.concatenate([dst, pad])
    src2 = jnp.concatenate([src_p, src_p + NP]).reshape(2 * NBLK, EBLK)
    dst2 = dst_p.reshape(NBLK, EBLK)
    hist_idx = jnp.concatenate([src_p, dst_p])

    ones128 = jnp.ones((EBLK, 128), f32)
    zeros128 = jnp.zeros((EBLK, 128), f32)

    cnt = _hist(hist_idx, ones128, zeros128)
    dsrc, ddst = _tc_norms(cnt)

    h_pad = jnp.concatenate([h, jnp.zeros((NP - N, h.shape[1]), f32)])
    W1r, bar, bbr = W1, ba.reshape(1, -1), bb.reshape(1, -1)
    b1r, b2r = b1.reshape(1, -1), b2.reshape(1, -1)

    def spmm128(y3):
        g = _spmm128(y3.reshape(2 * NP, 128), src2, dst2, zeros128)
        return g.reshape(2, NP, 128)

    dts = t_span[1:] - t_span[:-1]

    y1 = _tc_in(h_pad, W1r, b1r, dsrc)
    g1 = spmm128(y1)
    z, ya = _tc_first(g1, y1, ddst, Wa, bar, dsrc)

    n_steps = t_span.shape[0] - 1
    yo = None
    for i in range(n_steps):
        dt = dts[i]
        ks = []
        for s_idx in range(4):
            ga = spmm128(ya)
            yb = _tc_b(ga, ya, ddst, Wb, bbr, dsrc)
            gb = spmm128(yb)
            if s_idx < 3:
                coeff = 0.5 if s_idx < 2 else 1.0
                dtc = (coeff * dt).reshape(1, 1)
                k, ya = _tc_c_mid(gb, yb, z, ddst, Wa, bar, dsrc, dtc)
                ks.append(k)
            else:
                dt6 = (dt / 6.0).reshape(1, 1)
                if i < n_steps - 1:
                    z, ya = _tc_c_last(gb, yb, z, ks[0], ks[1], ks[2],
                                       ddst, Wa, bar, dsrc, dt6, True)
                else:
                    z, yo = _tc_c_last(gb, yb, z, ks[0], ks[1], ks[2],
                                       ddst, W2, b2r, dsrc, dt6, False)

    go = _spmm_es(yo, src_p, dst_p, zeros128)
    go = go.reshape(2, NP, 128)
    out = _tc_out(go, yo, ddst)
    return out[:N]


# speedup vs baseline: 3.4090x; 1.0023x over previous
import functools

import jax
import jax.numpy as jnp
from jax import lax
from jax.experimental import pallas as pl
from jax.experimental.pallas import tpu as pltpu
from jax.experimental.pallas import tpu_sc as plsc

N = 10000
NP = 10240
E = 320000
EP = 327680
EBLK = 128
NBLK = EP // EBLK
PAD_NODE = 10200
RB = 256
GRID = NP // RB
HIGH = jax.lax.Precision.HIGHEST

_mesh = plsc.VectorSubcoreMesh(core_axis_name="c", subcore_axis_name="s")


def _make_hist():
    per_tile = NBLK // 16

    @functools.partial(
        pl.kernel,
        out_type=jax.ShapeDtypeStruct((2 * NP, 128), jnp.float32),
        mesh=_mesh,
        scratch_types=[
            pltpu.VMEM((1, EBLK), jnp.int32),
            pltpu.VMEM((EBLK, 128), jnp.float32),
            pltpu.VMEM((EBLK, 128), jnp.float32),
            pltpu.VMEM_SHARED((NP, 128), jnp.float32),
        ],
    )
    def hist(hidx_hbm, ones_hbm, zeros_hbm, cnt_hbm, idx, ones_v, tmp, acc):
        c = lax.axis_index("c")
        s = lax.axis_index("s")
        pltpu.sync_copy(ones_hbm, ones_v)
        pltpu.sync_copy(zeros_hbm, tmp)
        rows_per_tile = NP // 16

        @pl.loop(0, rows_per_tile // EBLK)
        def _(j):
            pltpu.sync_copy(tmp, acc.at[pl.ds(s * rows_per_tile + j * EBLK, EBLK)])

        plsc.subcore_barrier()

        @pl.loop(0, per_tile)
        def _(b):
            e0 = (s * per_tile + b) * EBLK
            pltpu.sync_copy(hidx_hbm.at[pl.ds(c * EP + e0, EBLK)], idx.at[0])
            pltpu.sync_copy(ones_v, acc.at[idx.at[0]], add=True)

        plsc.subcore_barrier()

        @pl.loop(0, rows_per_tile // EBLK)
        def _(j):
            base = s * rows_per_tile + j * EBLK
            pltpu.sync_copy(acc.at[pl.ds(base, EBLK)], tmp)
            pltpu.sync_copy(tmp, cnt_hbm.at[pl.ds(c * NP + base, EBLK)])

    return hist


def _make_spmm(hw):
    per_tile = NBLK // 16
    rows_per_tile = NP // 16
    ICH = 16
    n_chunks = per_tile // ICH

    @functools.partial(
        pl.kernel,
        out_type=jax.ShapeDtypeStruct((2 * NP, hw), jnp.float32),
        mesh=_mesh,
        scratch_types=[
            pltpu.VMEM((ICH, EBLK), jnp.int32),
            pltpu.VMEM((ICH, EBLK), jnp.int32),
            pltpu.VMEM((EBLK, hw), jnp.float32),
            pltpu.VMEM((EBLK, hw), jnp.float32),
            pltpu.VMEM_SHARED((NP, hw), jnp.float32),
            pltpu.SemaphoreType.DMA,
            pltpu.SemaphoreType.DMA,
            pltpu.SemaphoreType.DMA,
        ],
    )
    def spmm(y_hbm, src2_hbm, dst_hbm, zeros_hbm, g_hbm,
             sidx, didx, rows0, rows1, acc, sem0, sem1, ssem):
        c = lax.axis_index("c")
        s = lax.axis_index("s")
        blk0 = s * per_tile
        pltpu.sync_copy(zeros_hbm, rows0)

        @pl.loop(0, rows_per_tile // EBLK)
        def _(j):
            pltpu.sync_copy(rows0, acc.at[pl.ds(s * rows_per_tile + j * EBLK, EBLK)])

        plsc.subcore_barrier()

        bufs = (rows0, rows1)
        gsems = (sem0, sem1)

        @pl.loop(0, n_chunks)
        def _(ch):
            pltpu.sync_copy(src2_hbm.at[pl.ds(c * NBLK + blk0 + ch * ICH, ICH)],
                            sidx)
            pltpu.sync_copy(dst_hbm.at[pl.ds(blk0 + ch * ICH, ICH)], didx)
            pltpu.async_copy(y_hbm.at[sidx.at[0]], rows0, sem0)

            @pl.loop(0, ICH // 2)
            def _(p):
                for par in range(2):
                    b = 2 * p + par
                    buf, gsem = bufs[par], gsems[par]
                    nbuf, ngsem = bufs[1 - par], gsems[1 - par]
                    pltpu.make_async_copy(y_hbm.at[sidx.at[0]], buf, gsem).wait()
                    pltpu.async_copy(buf, acc.at[didx.at[b]], ssem, add=True)

                    @pl.when(b >= 1)
                    def _():
                        pltpu.make_async_copy(zeros_hbm, nbuf, ssem).wait()

                    @pl.when(b + 1 < ICH)
                    def _():
                        pltpu.async_copy(y_hbm.at[sidx.at[b + 1]], nbuf, ngsem)

            pltpu.make_async_copy(zeros_hbm, rows1, ssem).wait()

        plsc.subcore_barrier()

        @pl.loop(0, rows_per_tile // EBLK)
        def _(j):
            base = s * rows_per_tile + j * EBLK
            pltpu.sync_copy(acc.at[pl.ds(base, EBLK)], rows0)
            pltpu.sync_copy(rows0, g_hbm.at[pl.ds(c * NP + base, EBLK)])

    return spmm


def _make_spmm_edgesplit():
    per_tile = NBLK // 32
    rows_per_tile = NP // 16

    @functools.partial(
        pl.kernel,
        out_type=jax.ShapeDtypeStruct((2 * NP, 128), jnp.float32),
        mesh=_mesh,
        scratch_types=[
            pltpu.VMEM((1, EBLK), jnp.int32),
            pltpu.VMEM((1, EBLK), jnp.int32),
            pltpu.VMEM((EBLK, 128), jnp.float32),
            pltpu.VMEM((EBLK, 128), jnp.float32),
            pltpu.VMEM_SHARED((NP, 128), jnp.float32),
        ],
    )
    def spmm(y_hbm, src_hbm, dst_hbm, zeros_hbm, g_hbm, sidx, didx, rows, tmp, acc):
        c = lax.axis_index("c")
        s = lax.axis_index("s")
        pltpu.sync_copy(zeros_hbm, tmp)

        @pl.loop(0, rows_per_tile // EBLK)
        def _(j):
            pltpu.sync_copy(tmp, acc.at[pl.ds(s * rows_per_tile + j * EBLK, EBLK)])

        plsc.subcore_barrier()

        @pl.loop(0, per_tile)
        def _(b):
            e0 = ((c * 16 + s) * per_tile + b) * EBLK
            pltpu.sync_copy(src_hbm.at[pl.ds(e0, EBLK)], sidx.at[0])
            pltpu.sync_copy(dst_hbm.at[pl.ds(e0, EBLK)], didx.at[0])
            pltpu.sync_copy(y_hbm.at[sidx.at[0]], rows)
            pltpu.sync_copy(rows, acc.at[didx.at[0]], add=True)

        plsc.subcore_barrier()

        @pl.loop(0, rows_per_tile // EBLK)
        def _(j):
            base = s * rows_per_tile + j * EBLK
            pltpu.sync_copy(acc.at[pl.ds(base, EBLK)], tmp)
            pltpu.sync_copy(tmp, g_hbm.at[pl.ds(c * NP + base, EBLK)])

    return spmm


_hist = _make_hist()
_spmm128 = _make_spmm(128)
_spmm_es = _make_spmm_edgesplit()


def _softplus(v):
    return jnp.maximum(v, 0.0) + jnp.log1p(jnp.exp(-jnp.abs(v)))


def _dot(x, w):
    return jnp.dot(x, w, precision=HIGH, preferred_element_type=jnp.float32)


def _split3(shape_hw):
    return pl.BlockSpec((2, RB, shape_hw), lambda i: (0, i, 0))


_spec_n1 = pl.BlockSpec((RB, 1), lambda i: (i, 0))


def _spec_full(shape):
    return pl.BlockSpec(shape, lambda i: tuple(0 for _ in shape))


def _cat(a, b):
    return jnp.concatenate([a, b], axis=1)


def _split_write(ref, val, hw):
    ref[0] = val[:, :hw]
    ref[1] = val[:, hw:]


def _norm_body(cnt_ref, dsrc_ref, ddst_ref):
    cnt = cnt_ref[...]
    dsrc = cnt[0:NP, 0:1]
    ddst = cnt[NP:2 * NP, 0:1]
    rows = lax.broadcasted_iota(jnp.int32, (NP, 1), 0)
    valid = rows < N
    dsrc_ref[...] = jnp.where(valid, lax.rsqrt(dsrc + 1.0), 0.0)
    ddst_ref[...] = jnp.where(valid, lax.rsqrt(ddst + 1.0), 0.0)


def _tc_norms(cnt):
    return pl.pallas_call(
        _norm_body,
        grid=(1,),
        in_specs=[_spec_full((2 * NP, 128))],
        out_specs=[pl.BlockSpec((NP, 1), lambda i: (0, 0))] * 2,
        out_shape=[jax.ShapeDtypeStruct((NP, 1), jnp.float32)] * 2,
    )(cnt)


def _in_body(x_ref, w_ref, b_ref, ds_ref, y_ref):
    y = (_dot(x_ref[...], w_ref[...]) + b_ref[...]) * ds_ref[...]
    _split_write(y_ref, y, 128)


def _tc_in(h_pad, W1, b1, dsrc):
    return pl.pallas_call(
        _in_body,
        grid=(GRID,),
        in_specs=[
            pl.BlockSpec((RB, 128), lambda i: (i, 0)),
            _spec_full((128, 256)),
            _spec_full((1, 256)),
            _spec_n1,
        ],
        out_specs=_split3(128),
        out_shape=jax.ShapeDtypeStruct((2, NP, 128), jnp.float32),
    )(h_pad, W1, b1, dsrc)


def _first_body(g_ref, y_ref, dd_ref, w_ref, b_ref, ds_ref, z_ref, ya_ref):
    dd = dd_ref[...]
    z0 = dd * (g_ref[0] + y_ref[0])
    z1 = dd * (g_ref[1] + y_ref[1])
    z_ref[0] = z0
    z_ref[1] = z1
    ya = (_dot(_cat(z0, z1), w_ref[...]) + b_ref[...]) * ds_ref[...]
    _split_write(ya_ref, ya, 128)


def _tc_first(g, y, ddst, Wa, ba, dsrc):
    return pl.pallas_call(
        _first_body,
        grid=(GRID,),
        in_specs=[
            _split3(128), _split3(128), _spec_n1,
            _spec_full((256, 256)), _spec_full((1, 256)), _spec_n1,
        ],
        out_specs=[_split3(128), _split3(128)],
        out_shape=[jax.ShapeDtypeStruct((2, NP, 128), jnp.float32)] * 2,
    )(g, y, ddst, Wa, ba, dsrc)


def _b_body(g_ref, y_ref, dd_ref, w_ref, b_ref, ds_ref, yb_ref):
    dd = dd_ref[...]
    x0 = dd * (g_ref[0] + y_ref[0])
    x1 = dd * (g_ref[1] + y_ref[1])
    x = _softplus(_cat(x0, x1))
    yb = (_dot(x, w_ref[...]) + b_ref[...]) * ds_ref[...]
    _split_write(yb_ref, yb, 128)


def _tc_b(g, y, ddst, Wb, bb, dsrc):
    return pl.pallas_call(
        _b_body,
        grid=(GRID,),
        in_specs=[
            _split3(128), _split3(128), _spec_n1,
            _spec_full((256, 256)), _spec_full((1, 256)), _spec_n1,
        ],
        out_specs=_split3(128),
        out_shape=jax.ShapeDtypeStruct((2, NP, 128), jnp.float32),
    )(g, y, ddst, Wb, bb, dsrc)


def _c_mid_body(g_ref, y_ref, z_ref, dd_ref, w_ref, b_ref, ds_ref, dtc_ref,
                k_ref, ya_ref):
    dd = dd_ref[...]
    dtc = dtc_ref[0, 0]
    k0 = dd * (g_ref[0] + y_ref[0])
    k1 = dd * (g_ref[1] + y_ref[1])
    k_ref[0] = k0
    k_ref[1] = k1
    u = _cat(z_ref[0] + dtc * k0, z_ref[1] + dtc * k1)
    ya = (_dot(u, w_ref[...]) + b_ref[...]) * ds_ref[...]
    _split_write(ya_ref, ya, 128)


def _tc_c_mid(g, y, z, ddst, Wa, ba, dsrc, dtc):
    return pl.pallas_call(
        _c_mid_body,
        grid=(GRID,),
        in_specs=[
            _split3(128), _split3(128), _split3(128), _spec_n1,
            _spec_full((256, 256)), _spec_full((1, 256)), _spec_n1,
            _spec_full((1, 1)),
        ],
        out_specs=[_split3(128), _split3(128)],
        out_shape=[jax.ShapeDtypeStruct((2, NP, 128), jnp.float32)] * 2,
    )(g, y, z, ddst, Wa, ba, dsrc, dtc)


def _c_last_body(g_ref, y_ref, z_ref, k1_ref, k2_ref, k3_ref, dd_ref,
                 w_ref, b_ref, ds_ref, dt6_ref, zn_ref, ya_ref, *, split_out):
    dd = dd_ref[...]
    dt6 = dt6_ref[0, 0]
    zn = []
    for hh in range(2):
        k4 = dd * (g_ref[hh] + y_ref[hh])
        znh = z_ref[hh] + dt6 * (k1_ref[hh] + 2.0 * k2_ref[hh]
                                 + 2.0 * k3_ref[hh] + k4)
        zn_ref[hh] = znh
        zn.append(znh)
    ya = (_dot(_cat(zn[0], zn[1]), w_ref[...]) + b_ref[...]) * ds_ref[...]
    if split_out:
        _split_write(ya_ref, ya, 128)
    else:
        ya_ref[...] = ya


def _tc_c_last(g, y, z, k1, k2, k3, ddst, W, b, dsrc, dt6, split_out):
    wout = W.shape[1]
    if split_out:
        ya_spec = _split3(128)
        ya_shape = jax.ShapeDtypeStruct((2, NP, 128), jnp.float32)
    else:
        ya_spec = pl.BlockSpec((RB, wout), lambda i: (i, 0))
        ya_shape = jax.ShapeDtypeStruct((NP, wout), jnp.float32)
    return pl.pallas_call(
        functools.partial(_c_last_body, split_out=split_out),
        grid=(GRID,),
        in_specs=[
            _split3(128), _split3(128), _split3(128), _split3(128),
            _split3(128), _split3(128), _spec_n1,
            _spec_full((256, wout)), _spec_full((1, wout)), _spec_n1,
            _spec_full((1, 1)),
        ],
        out_specs=[_split3(128), ya_spec],
        out_shape=[jax.ShapeDtypeStruct((2, NP, 128), jnp.float32), ya_shape],
    )(g, y, z, k1, k2, k3, ddst, W, b, dsrc, dt6)


def _out_body(g_ref, y_ref, dd_ref, o_ref):
    o_ref[...] = dd_ref[...] * (g_ref[0] + g_ref[1] + y_ref[...])


def _tc_out(g, y, ddst):
    return pl.pallas_call(
        _out_body,
        grid=(GRID,),
        in_specs=[_split3(128), pl.BlockSpec((RB, 128), lambda i: (i, 0)),
                  _spec_n1],
        out_specs=pl.BlockSpec((RB, 128), lambda i: (i, 0)),
        out_shape=jax.ShapeDtypeStruct((NP, 128), jnp.float32),
    )(g, y, ddst)


def kernel(h, edge_index, t_span, W1, b1, Wa, ba, Wb, bb, W2, b2):
    f32 = jnp.float32
    src = edge_index[0]
    dst = edge_index[1]
    pad = jnp.full((EP - E,), PAD_NODE, jnp.int32)
    src_p = jnp.concatenate([src, pad])
    dst_p = jnp.concatenate([dst, pad])
    src2 = jnp.concatenate([src_p, src_p + NP]).reshape(2 * NBLK, EBLK)
    dst2 = dst_p.reshape(NBLK, EBLK)
    hist_idx = jnp.concatenate([src_p, dst_p])

    ones128 = jnp.ones((EBLK, 128), f32)
    zeros128 = jnp.zeros((EBLK, 128), f32)

    cnt = _hist(hist_idx, ones128, zeros128)
    dsrc, ddst = _tc_norms(cnt)

    h_pad = jnp.concatenate([h, jnp.zeros((NP - N, h.shape[1]), f32)])
    W1r, bar, bbr = W1, ba.reshape(1, -1), bb.reshape(1, -1)
    b1r, b2r = b1.reshape(1, -1), b2.reshape(1, -1)

    def spmm128(y3):
        g = _spmm128(y3.reshape(2 * NP, 128), src2, dst2, zeros128)
        return g.reshape(2, NP, 128)

    dts = t_span[1:] - t_span[:-1]

    y1 = _tc_in(h_pad, W1r, b1r, dsrc)
    g1 = spmm128(y1)
    z, ya = _tc_first(g1, y1, ddst, Wa, bar, dsrc)

    n_steps = t_span.shape[0] - 1
    yo = None
    for i in range(n_steps):
        dt = dts[i]
        ks = []
        for s_idx in range(4):
            ga = spmm128(ya)
            yb = _tc_b(ga, ya, ddst, Wb, bbr, dsrc)
            gb = spmm128(yb)
            if s_idx < 3:
                coeff = 0.5 if s_idx < 2 else 1.0
                dtc = (coeff * dt).reshape(1, 1)
                k, ya = _tc_c_mid(gb, yb, z, ddst, Wa, bar, dsrc, dtc)
                ks.append(k)
            else:
                dt6 = (dt / 6.0).reshape(1, 1)
                if i < n_steps - 1:
                    z, ya = _tc_c_last(gb, yb, z, ks[0], ks[1], ks[2],
                                       ddst, Wa, bar, dsrc, dt6, True)
                else:
                    z, yo = _tc_c_last(gb, yb, z, ks[0], ks[1], ks[2],
                                       ddst, W2, b2r, dsrc, dt6, False)

    go = _spmm_es(yo, src_p, dst_p, zeros128)
    go = go.reshape(2, NP, 128)
    out = _tc_out(go, yo, ddst)
    return out[:N]


# speedup vs baseline: 3.6638x; 1.0747x over previous
import functools

import jax
import jax.numpy as jnp
from jax import lax
from jax.experimental import pallas as pl
from jax.experimental.pallas import tpu as pltpu
from jax.experimental.pallas import tpu_sc as plsc

N = 10000
NP = 10240
E = 320000
EP = 327680
EBLK = 128
NBLK = EP // EBLK
PAD_NODE = 10200
RB = 256
GRID = NP // RB
HIGH = jax.lax.Precision.HIGHEST

_mesh = plsc.VectorSubcoreMesh(core_axis_name="c", subcore_axis_name="s")


def _make_hist():
    per_tile = NBLK // 16

    @functools.partial(
        pl.kernel,
        out_type=jax.ShapeDtypeStruct((2 * NP, 128), jnp.float32),
        mesh=_mesh,
        scratch_types=[
            pltpu.VMEM((1, EBLK), jnp.int32),
            pltpu.VMEM((EBLK, 128), jnp.float32),
            pltpu.VMEM((EBLK, 128), jnp.float32),
            pltpu.VMEM_SHARED((NP, 128), jnp.float32),
        ],
    )
    def hist(hidx_hbm, ones_hbm, zeros_hbm, cnt_hbm, idx, ones_v, tmp, acc):
        c = lax.axis_index("c")
        s = lax.axis_index("s")
        pltpu.sync_copy(ones_hbm, ones_v)
        pltpu.sync_copy(zeros_hbm, tmp)
        rows_per_tile = NP // 16

        @pl.loop(0, rows_per_tile // EBLK)
        def _(j):
            pltpu.sync_copy(tmp, acc.at[pl.ds(s * rows_per_tile + j * EBLK, EBLK)])

        plsc.subcore_barrier()

        @pl.loop(0, per_tile)
        def _(b):
            e0 = (s * per_tile + b) * EBLK
            pltpu.sync_copy(hidx_hbm.at[pl.ds(c * EP + e0, EBLK)], idx.at[0])
            pltpu.sync_copy(ones_v, acc.at[idx.at[0]], add=True)

        plsc.subcore_barrier()

        @pl.loop(0, rows_per_tile // EBLK)
        def _(j):
            base = s * rows_per_tile + j * EBLK
            pltpu.sync_copy(acc.at[pl.ds(base, EBLK)], tmp)
            pltpu.sync_copy(tmp, cnt_hbm.at[pl.ds(c * NP + base, EBLK)])

    return hist


def _make_spmm(hw):
    per_tile = NBLK // 16
    rows_per_tile = NP // 16
    ICH = 16
    n_chunks = per_tile // ICH

    @functools.partial(
        pl.kernel,
        out_type=jax.ShapeDtypeStruct((2 * NP, hw), jnp.float32),
        mesh=_mesh,
        scratch_types=[
            pltpu.VMEM((ICH, EBLK), jnp.int32),
            pltpu.VMEM((ICH, EBLK), jnp.int32),
            pltpu.VMEM((EBLK, hw), jnp.float32),
            pltpu.VMEM((EBLK, hw), jnp.float32),
            pltpu.VMEM_SHARED((NP, hw), jnp.float32),
            pltpu.SemaphoreType.DMA,
            pltpu.SemaphoreType.DMA,
            pltpu.SemaphoreType.DMA,
        ],
    )
    def spmm(y_hbm, src2_hbm, dst_hbm, zeros_hbm, g_hbm,
             sidx, didx, rows0, rows1, acc, sem0, sem1, ssem):
        c = lax.axis_index("c")
        s = lax.axis_index("s")
        blk0 = s * per_tile
        pltpu.sync_copy(zeros_hbm, rows0)

        @pl.loop(0, rows_per_tile // EBLK)
        def _(j):
            pltpu.sync_copy(rows0, acc.at[pl.ds(s * rows_per_tile + j * EBLK, EBLK)])

        plsc.subcore_barrier()

        bufs = (rows0, rows1)
        gsems = (sem0, sem1)

        @pl.loop(0, n_chunks)
        def _(ch):
            pltpu.sync_copy(src2_hbm.at[pl.ds(c * NBLK + blk0 + ch * ICH, ICH)],
                            sidx)
            pltpu.sync_copy(dst_hbm.at[pl.ds(blk0 + ch * ICH, ICH)], didx)
            pltpu.async_copy(y_hbm.at[sidx.at[0]], rows0, sem0)

            @pl.loop(0, ICH // 2)
            def _(p):
                for par in range(2):
                    b = 2 * p + par
                    buf, gsem = bufs[par], gsems[par]
                    nbuf, ngsem = bufs[1 - par], gsems[1 - par]

                    @pl.when(b >= 1)
                    def _():
                        pltpu.make_async_copy(zeros_hbm, nbuf, ssem).wait()

                    @pl.when(b + 1 < ICH)
                    def _():
                        pltpu.async_copy(y_hbm.at[sidx.at[b + 1]], nbuf, ngsem)

                    pltpu.make_async_copy(y_hbm.at[sidx.at[0]], buf, gsem).wait()
                    pltpu.async_copy(buf, acc.at[didx.at[b]], ssem, add=True)

            pltpu.make_async_copy(zeros_hbm, rows1, ssem).wait()

        plsc.subcore_barrier()

        @pl.loop(0, rows_per_tile // EBLK)
        def _(j):
            base = s * rows_per_tile + j * EBLK
            pltpu.sync_copy(acc.at[pl.ds(base, EBLK)], rows0)
            pltpu.sync_copy(rows0, g_hbm.at[pl.ds(c * NP + base, EBLK)])

    return spmm


def _make_spmm_edgesplit():
    per_tile = NBLK // 32
    rows_per_tile = NP // 16

    @functools.partial(
        pl.kernel,
        out_type=jax.ShapeDtypeStruct((2 * NP, 128), jnp.float32),
        mesh=_mesh,
        scratch_types=[
            pltpu.VMEM((1, EBLK), jnp.int32),
            pltpu.VMEM((1, EBLK), jnp.int32),
            pltpu.VMEM((EBLK, 128), jnp.float32),
            pltpu.VMEM((EBLK, 128), jnp.float32),
            pltpu.VMEM_SHARED((NP, 128), jnp.float32),
        ],
    )
    def spmm(y_hbm, src_hbm, dst_hbm, zeros_hbm, g_hbm, sidx, didx, rows, tmp, acc):
        c = lax.axis_index("c")
        s = lax.axis_index("s")
        pltpu.sync_copy(zeros_hbm, tmp)

        @pl.loop(0, rows_per_tile // EBLK)
        def _(j):
            pltpu.sync_copy(tmp, acc.at[pl.ds(s * rows_per_tile + j * EBLK, EBLK)])

        plsc.subcore_barrier()

        @pl.loop(0, per_tile)
        def _(b):
            e0 = ((c * 16 + s) * per_tile + b) * EBLK
            pltpu.sync_copy(src_hbm.at[pl.ds(e0, EBLK)], sidx.at[0])
            pltpu.sync_copy(dst_hbm.at[pl.ds(e0, EBLK)], didx.at[0])
            pltpu.sync_copy(y_hbm.at[sidx.at[0]], rows)
            pltpu.sync_copy(rows, acc.at[didx.at[0]], add=True)

        plsc.subcore_barrier()

        @pl.loop(0, rows_per_tile // EBLK)
        def _(j):
            base = s * rows_per_tile + j * EBLK
            pltpu.sync_copy(acc.at[pl.ds(base, EBLK)], tmp)
            pltpu.sync_copy(tmp, g_hbm.at[pl.ds(c * NP + base, EBLK)])

    return spmm


_hist = _make_hist()
_spmm128 = _make_spmm(128)
_spmm_es = _make_spmm_edgesplit()


def _softplus(v):
    return jnp.maximum(v, 0.0) + jnp.log1p(jnp.exp(-jnp.abs(v)))


def _dot(x, w):
    return jnp.dot(x, w, precision=HIGH, preferred_element_type=jnp.float32)


def _split3(shape_hw):
    return pl.BlockSpec((2, RB, shape_hw), lambda i: (0, i, 0))


_spec_n1 = pl.BlockSpec((RB, 1), lambda i: (i, 0))


def _spec_full(shape):
    return pl.BlockSpec(shape, lambda i: tuple(0 for _ in shape))


def _cat(a, b):
    return jnp.concatenate([a, b], axis=1)


def _split_write(ref, val, hw):
    ref[0] = val[:, :hw]
    ref[1] = val[:, hw:]


def _norm_body(cnt_ref, dsrc_ref, ddst_ref):
    cnt = cnt_ref[...]
    dsrc = cnt[0:NP, 0:1]
    ddst = cnt[NP:2 * NP, 0:1]
    rows = lax.broadcasted_iota(jnp.int32, (NP, 1), 0)
    valid = rows < N
    dsrc_ref[...] = jnp.where(valid, lax.rsqrt(dsrc + 1.0), 0.0)
    ddst_ref[...] = jnp.where(valid, lax.rsqrt(ddst + 1.0), 0.0)


def _tc_norms(cnt):
    return pl.pallas_call(
        _norm_body,
        grid=(1,),
        in_specs=[_spec_full((2 * NP, 128))],
        out_specs=[pl.BlockSpec((NP, 1), lambda i: (0, 0))] * 2,
        out_shape=[jax.ShapeDtypeStruct((NP, 1), jnp.float32)] * 2,
    )(cnt)


def _in_body(x_ref, w_ref, b_ref, ds_ref, y_ref):
    y = (_dot(x_ref[...], w_ref[...]) + b_ref[...]) * ds_ref[...]
    _split_write(y_ref, y, 128)


def _tc_in(h_pad, W1, b1, dsrc):
    return pl.pallas_call(
        _in_body,
        grid=(GRID,),
        in_specs=[
            pl.BlockSpec((RB, 128), lambda i: (i, 0)),
            _spec_full((128, 256)),
            _spec_full((1, 256)),
            _spec_n1,
        ],
        out_specs=_split3(128),
        out_shape=jax.ShapeDtypeStruct((2, NP, 128), jnp.float32),
    )(h_pad, W1, b1, dsrc)


def _first_body(g_ref, y_ref, dd_ref, w_ref, b_ref, ds_ref, z_ref, ya_ref):
    dd = dd_ref[...]
    z0 = dd * (g_ref[0] + y_ref[0])
    z1 = dd * (g_ref[1] + y_ref[1])
    z_ref[0] = z0
    z_ref[1] = z1
    ya = (_dot(_cat(z0, z1), w_ref[...]) + b_ref[...]) * ds_ref[...]
    _split_write(ya_ref, ya, 128)


def _tc_first(g, y, ddst, Wa, ba, dsrc):
    return pl.pallas_call(
        _first_body,
        grid=(GRID,),
        in_specs=[
            _split3(128), _split3(128), _spec_n1,
            _spec_full((256, 256)), _spec_full((1, 256)), _spec_n1,
        ],
        out_specs=[_split3(128), _split3(128)],
        out_shape=[jax.ShapeDtypeStruct((2, NP, 128), jnp.float32)] * 2,
    )(g, y, ddst, Wa, ba, dsrc)


def _b_body(g_ref, y_ref, dd_ref, w_ref, b_ref, ds_ref, yb_ref):
    dd = dd_ref[...]
    x0 = dd * (g_ref[0] + y_ref[0])
    x1 = dd * (g_ref[1] + y_ref[1])
    x = _softplus(_cat(x0, x1))
    yb = (_dot(x, w_ref[...]) + b_ref[...]) * ds_ref[...]
    _split_write(yb_ref, yb, 128)


def _tc_b(g, y, ddst, Wb, bb, dsrc):
    return pl.pallas_call(
        _b_body,
        grid=(GRID,),
        in_specs=[
            _split3(128), _split3(128), _spec_n1,
            _spec_full((256, 256)), _spec_full((1, 256)), _spec_n1,
        ],
        out_specs=_split3(128),
        out_shape=jax.ShapeDtypeStruct((2, NP, 128), jnp.float32),
    )(g, y, ddst, Wb, bb, dsrc)


def _c_mid_body(g_ref, y_ref, z_ref, dd_ref, w_ref, b_ref, ds_ref, dtc_ref,
                k_ref, ya_ref):
    dd = dd_ref[...]
    dtc = dtc_ref[0, 0]
    k0 = dd * (g_ref[0] + y_ref[0])
    k1 = dd * (g_ref[1] + y_ref[1])
    k_ref[0] = k0
    k_ref[1] = k1
    u = _cat(z_ref[0] + dtc * k0, z_ref[1] + dtc * k1)
    ya = (_dot(u, w_ref[...]) + b_ref[...]) * ds_ref[...]
    _split_write(ya_ref, ya, 128)


def _tc_c_mid(g, y, z, ddst, Wa, ba, dsrc, dtc):
    return pl.pallas_call(
        _c_mid_body,
        grid=(GRID,),
        in_specs=[
            _split3(128), _split3(128), _split3(128), _spec_n1,
            _spec_full((256, 256)), _spec_full((1, 256)), _spec_n1,
            _spec_full((1, 1)),
        ],
        out_specs=[_split3(128), _split3(128)],
        out_shape=[jax.ShapeDtypeStruct((2, NP, 128), jnp.float32)] * 2,
    )(g, y, z, ddst, Wa, ba, dsrc, dtc)


def _c_last_body(g_ref, y_ref, z_ref, k1_ref, k2_ref, k3_ref, dd_ref,
                 w_ref, b_ref, ds_ref, dt6_ref, zn_ref, ya_ref, *, split_out):
    dd = dd_ref[...]
    dt6 = dt6_ref[0, 0]
    zn = []
    for hh in range(2):
        k4 = dd * (g_ref[hh] + y_ref[hh])
        znh = z_ref[hh] + dt6 * (k1_ref[hh] + 2.0 * k2_ref[hh]
                                 + 2.0 * k3_ref[hh] + k4)
        zn_ref[hh] = znh
        zn.append(znh)
    ya = (_dot(_cat(zn[0], zn[1]), w_ref[...]) + b_ref[...]) * ds_ref[...]
    if split_out:
        _split_write(ya_ref, ya, 128)
    else:
        ya_ref[...] = ya


def _tc_c_last(g, y, z, k1, k2, k3, ddst, W, b, dsrc, dt6, split_out):
    wout = W.shape[1]
    if split_out:
        ya_spec = _split3(128)
        ya_shape = jax.ShapeDtypeStruct((2, NP, 128), jnp.float32)
    else:
        ya_spec = pl.BlockSpec((RB, wout), lambda i: (i, 0))
        ya_shape = jax.ShapeDtypeStruct((NP, wout), jnp.float32)
    return pl.pallas_call(
        functools.partial(_c_last_body, split_out=split_out),
        grid=(GRID,),
        in_specs=[
            _split3(128), _split3(128), _split3(128), _split3(128),
            _split3(128), _split3(128), _spec_n1,
            _spec_full((256, wout)), _spec_full((1, wout)), _spec_n1,
            _spec_full((1, 1)),
        ],
        out_specs=[_split3(128), ya_spec],
        out_shape=[jax.ShapeDtypeStruct((2, NP, 128), jnp.float32), ya_shape],
    )(g, y, z, k1, k2, k3, ddst, W, b, dsrc, dt6)


def _out_body(g_ref, y_ref, dd_ref, o_ref):
    o_ref[...] = dd_ref[...] * (g_ref[0] + g_ref[1] + y_ref[...])


def _tc_out(g, y, ddst):
    return pl.pallas_call(
        _out_body,
        grid=(GRID,),
        in_specs=[_split3(128), pl.BlockSpec((RB, 128), lambda i: (i, 0)),
                  _spec_n1],
        out_specs=pl.BlockSpec((RB, 128), lambda i: (i, 0)),
        out_shape=jax.ShapeDtypeStruct((NP, 128), jnp.float32),
    )(g, y, ddst)


def kernel(h, edge_index, t_span, W1, b1, Wa, ba, Wb, bb, W2, b2):
    f32 = jnp.float32
    src = edge_index[0]
    dst = edge_index[1]
    pad = jnp.full((EP - E,), PAD_NODE, jnp.int32)
    src_p = jnp.concatenate([src, pad])
    dst_p = jnp.concatenate([dst, pad])
    src2 = jnp.concatenate([src_p, src_p + NP]).reshape(2 * NBLK, EBLK)
    dst2 = dst_p.reshape(NBLK, EBLK)
    hist_idx = jnp.concatenate([src_p, dst_p])

    ones128 = jnp.ones((EBLK, 128), f32)
    zeros128 = jnp.zeros((EBLK, 128), f32)

    cnt = _hist(hist_idx, ones128, zeros128)
    dsrc, ddst = _tc_norms(cnt)

    h_pad = jnp.concatenate([h, jnp.zeros((NP - N, h.shape[1]), f32)])
    W1r, bar, bbr = W1, ba.reshape(1, -1), bb.reshape(1, -1)
    b1r, b2r = b1.reshape(1, -1), b2.reshape(1, -1)

    def spmm128(y3):
        g = _spmm128(y3.reshape(2 * NP, 128), src2, dst2, zeros128)
        return g.reshape(2, NP, 128)

    dts = t_span[1:] - t_span[:-1]

    y1 = _tc_in(h_pad, W1r, b1r, dsrc)
    g1 = spmm128(y1)
    z, ya = _tc_first(g1, y1, ddst, Wa, bar, dsrc)

    n_steps = t_span.shape[0] - 1
    yo = None
    for i in range(n_steps):
        dt = dts[i]
        ks = []
        for s_idx in range(4):
            ga = spmm128(ya)
            yb = _tc_b(ga, ya, ddst, Wb, bbr, dsrc)
            gb = spmm128(yb)
            if s_idx < 3:
                coeff = 0.5 if s_idx < 2 else 1.0
                dtc = (coeff * dt).reshape(1, 1)
                k, ya = _tc_c_mid(gb, yb, z, ddst, Wa, bar, dsrc, dtc)
                ks.append(k)
            else:
                dt6 = (dt / 6.0).reshape(1, 1)
                if i < n_steps - 1:
                    z, ya = _tc_c_last(gb, yb, z, ks[0], ks[1], ks[2],
                                       ddst, Wa, bar, dsrc, dt6, True)
                else:
                    z, yo = _tc_c_last(gb, yb, z, ks[0], ks[1], ks[2],
                                       ddst, W2, b2r, dsrc, dt6, False)

    go = _spmm_es(yo, src_p, dst_p, zeros128)
    go = go.reshape(2, NP, 128)
    out = _tc_out(go, yo, ddst)
    return out[:N]


# speedup vs baseline: 3.7690x; 1.0287x over previous
import functools

import jax
import jax.numpy as jnp
from jax import lax
from jax.experimental import pallas as pl
from jax.experimental.pallas import tpu as pltpu
from jax.experimental.pallas import tpu_sc as plsc

N = 10000
NP = 10240
E = 320000
EP = 327680
EBLK = 128
NBLK = EP // EBLK
PAD_NODE = 10200
RB = 256
GRID = NP // RB
HIGH = jax.lax.Precision.HIGHEST

_mesh = plsc.VectorSubcoreMesh(core_axis_name="c", subcore_axis_name="s")


def _make_hist():
    per_tile = NBLK // 16

    @functools.partial(
        pl.kernel,
        out_type=jax.ShapeDtypeStruct((2 * NP, 128), jnp.float32),
        mesh=_mesh,
        scratch_types=[
            pltpu.VMEM((1, EBLK), jnp.int32),
            pltpu.VMEM((EBLK, 128), jnp.float32),
            pltpu.VMEM((EBLK, 128), jnp.float32),
            pltpu.VMEM_SHARED((NP, 128), jnp.float32),
        ],
    )
    def hist(hidx_hbm, ones_hbm, zeros_hbm, cnt_hbm, idx, ones_v, tmp, acc):
        c = lax.axis_index("c")
        s = lax.axis_index("s")
        pltpu.sync_copy(ones_hbm, ones_v)
        pltpu.sync_copy(zeros_hbm, tmp)
        rows_per_tile = NP // 16

        @pl.loop(0, rows_per_tile // EBLK)
        def _(j):
            pltpu.sync_copy(tmp, acc.at[pl.ds(s * rows_per_tile + j * EBLK, EBLK)])

        plsc.subcore_barrier()

        @pl.loop(0, per_tile)
        def _(b):
            e0 = (s * per_tile + b) * EBLK
            pltpu.sync_copy(hidx_hbm.at[pl.ds(c * EP + e0, EBLK)], idx.at[0])
            pltpu.sync_copy(ones_v, acc.at[idx.at[0]], add=True)

        plsc.subcore_barrier()

        @pl.loop(0, rows_per_tile // EBLK)
        def _(j):
            base = s * rows_per_tile + j * EBLK
            pltpu.sync_copy(acc.at[pl.ds(base, EBLK)], tmp)
            pltpu.sync_copy(tmp, cnt_hbm.at[pl.ds(c * NP + base, EBLK)])

    return hist


def _make_spmm(hw):
    per_tile = NBLK // 16
    rows_per_tile = NP // 16
    ICH = 40
    n_chunks = per_tile // ICH

    @functools.partial(
        pl.kernel,
        out_type=jax.ShapeDtypeStruct((2 * NP, hw), jnp.float32),
        mesh=_mesh,
        scratch_types=[
            pltpu.VMEM((ICH, EBLK), jnp.int32),
            pltpu.VMEM((ICH, EBLK), jnp.int32),
            pltpu.VMEM((EBLK, hw), jnp.float32),
            pltpu.VMEM((EBLK, hw), jnp.float32),
            pltpu.VMEM_SHARED((NP, hw), jnp.float32),
            pltpu.SemaphoreType.DMA,
            pltpu.SemaphoreType.DMA,
            pltpu.SemaphoreType.DMA,
        ],
    )
    def spmm(y_hbm, src2_hbm, dst_hbm, zeros_hbm, g_hbm,
             sidx, didx, rows0, rows1, acc, sem0, sem1, ssem):
        c = lax.axis_index("c")
        s = lax.axis_index("s")
        blk0 = s * per_tile
        pltpu.sync_copy(zeros_hbm, rows0)

        @pl.loop(0, rows_per_tile // EBLK)
        def _(j):
            pltpu.sync_copy(rows0, acc.at[pl.ds(s * rows_per_tile + j * EBLK, EBLK)])

        plsc.subcore_barrier()

        bufs = (rows0, rows1)
        gsems = (sem0, sem1)

        @pl.loop(0, n_chunks)
        def _(ch):
            pltpu.sync_copy(src2_hbm.at[pl.ds(c * NBLK + blk0 + ch * ICH, ICH)],
                            sidx)
            pltpu.sync_copy(dst_hbm.at[pl.ds(blk0 + ch * ICH, ICH)], didx)
            pltpu.async_copy(y_hbm.at[sidx.at[0]], rows0, sem0)

            @pl.loop(0, ICH // 2)
            def _(p):
                for par in range(2):
                    b = 2 * p + par
                    buf, gsem = bufs[par], gsems[par]
                    nbuf, ngsem = bufs[1 - par], gsems[1 - par]

                    @pl.when(b >= 1)
                    def _():
                        pltpu.make_async_copy(zeros_hbm, nbuf, ssem).wait()

                    @pl.when(b + 1 < ICH)
                    def _():
                        pltpu.async_copy(y_hbm.at[sidx.at[b + 1]], nbuf, ngsem)

                    pltpu.make_async_copy(y_hbm.at[sidx.at[0]], buf, gsem).wait()
                    pltpu.async_copy(buf, acc.at[didx.at[b]], ssem, add=True)

            pltpu.make_async_copy(zeros_hbm, rows1, ssem).wait()

        plsc.subcore_barrier()

        @pl.loop(0, rows_per_tile // EBLK)
        def _(j):
            base = s * rows_per_tile + j * EBLK
            pltpu.sync_copy(acc.at[pl.ds(base, EBLK)], rows0)
            pltpu.sync_copy(rows0, g_hbm.at[pl.ds(c * NP + base, EBLK)])

    return spmm


def _make_spmm_edgesplit():
    per_tile = NBLK // 32
    rows_per_tile = NP // 16

    @functools.partial(
        pl.kernel,
        out_type=jax.ShapeDtypeStruct((2 * NP, 128), jnp.float32),
        mesh=_mesh,
        scratch_types=[
            pltpu.VMEM((1, EBLK), jnp.int32),
            pltpu.VMEM((1, EBLK), jnp.int32),
            pltpu.VMEM((EBLK, 128), jnp.float32),
            pltpu.VMEM((EBLK, 128), jnp.float32),
            pltpu.VMEM_SHARED((NP, 128), jnp.float32),
        ],
    )
    def spmm(y_hbm, src_hbm, dst_hbm, zeros_hbm, g_hbm, sidx, didx, rows, tmp, acc):
        c = lax.axis_index("c")
        s = lax.axis_index("s")
        pltpu.sync_copy(zeros_hbm, tmp)

        @pl.loop(0, rows_per_tile // EBLK)
        def _(j):
            pltpu.sync_copy(tmp, acc.at[pl.ds(s * rows_per_tile + j * EBLK, EBLK)])

        plsc.subcore_barrier()

        @pl.loop(0, per_tile)
        def _(b):
            e0 = ((c * 16 + s) * per_tile + b) * EBLK
            pltpu.sync_copy(src_hbm.at[pl.ds(e0, EBLK)], sidx.at[0])
            pltpu.sync_copy(dst_hbm.at[pl.ds(e0, EBLK)], didx.at[0])
            pltpu.sync_copy(y_hbm.at[sidx.at[0]], rows)
            pltpu.sync_copy(rows, acc.at[didx.at[0]], add=True)

        plsc.subcore_barrier()

        @pl.loop(0, rows_per_tile // EBLK)
        def _(j):
            base = s * rows_per_tile + j * EBLK
            pltpu.sync_copy(acc.at[pl.ds(base, EBLK)], tmp)
            pltpu.sync_copy(tmp, g_hbm.at[pl.ds(c * NP + base, EBLK)])

    return spmm


_hist = _make_hist()
_spmm128 = _make_spmm(128)
_spmm_es = _make_spmm_edgesplit()


def _softplus(v):
    return jnp.maximum(v, 0.0) + jnp.log1p(jnp.exp(-jnp.abs(v)))


def _dot(x, w):
    return jnp.dot(x, w, precision=HIGH, preferred_element_type=jnp.float32)


def _split3(shape_hw):
    return pl.BlockSpec((2, RB, shape_hw), lambda i: (0, i, 0))


_spec_n1 = pl.BlockSpec((RB, 1), lambda i: (i, 0))


def _spec_full(shape):
    return pl.BlockSpec(shape, lambda i: tuple(0 for _ in shape))


def _cat(a, b):
    return jnp.concatenate([a, b], axis=1)


def _split_write(ref, val, hw):
    ref[0] = val[:, :hw]
    ref[1] = val[:, hw:]


def _norm_body(cnt_ref, dsrc_ref, ddst_ref):
    cnt = cnt_ref[...]
    dsrc = cnt[0:NP, 0:1]
    ddst = cnt[NP:2 * NP, 0:1]
    rows = lax.broadcasted_iota(jnp.int32, (NP, 1), 0)
    valid = rows < N
    dsrc_ref[...] = jnp.where(valid, lax.rsqrt(dsrc + 1.0), 0.0)
    ddst_ref[...] = jnp.where(valid, lax.rsqrt(ddst + 1.0), 0.0)


def _tc_norms(cnt):
    return pl.pallas_call(
        _norm_body,
        grid=(1,),
        in_specs=[_spec_full((2 * NP, 128))],
        out_specs=[pl.BlockSpec((NP, 1), lambda i: (0, 0))] * 2,
        out_shape=[jax.ShapeDtypeStruct((NP, 1), jnp.float32)] * 2,
    )(cnt)


def _in_body(x_ref, w_ref, b_ref, ds_ref, y_ref):
    y = (_dot(x_ref[...], w_ref[...]) + b_ref[...]) * ds_ref[...]
    _split_write(y_ref, y, 128)


def _tc_in(h_pad, W1, b1, dsrc):
    return pl.pallas_call(
        _in_body,
        grid=(GRID,),
        in_specs=[
            pl.BlockSpec((RB, 128), lambda i: (i, 0)),
            _spec_full((128, 256)),
            _spec_full((1, 256)),
            _spec_n1,
        ],
        out_specs=_split3(128),
        out_shape=jax.ShapeDtypeStruct((2, NP, 128), jnp.float32),
    )(h_pad, W1, b1, dsrc)


def _first_body(g_ref, y_ref, dd_ref, w_ref, b_ref, ds_ref, z_ref, ya_ref):
    dd = dd_ref[...]
    z0 = dd * (g_ref[0] + y_ref[0])
    z1 = dd * (g_ref[1] + y_ref[1])
    z_ref[0] = z0
    z_ref[1] = z1
    ya = (_dot(_cat(z0, z1), w_ref[...]) + b_ref[...]) * ds_ref[...]
    _split_write(ya_ref, ya, 128)


def _tc_first(g, y, ddst, Wa, ba, dsrc):
    return pl.pallas_call(
        _first_body,
        grid=(GRID,),
        in_specs=[
            _split3(128), _split3(128), _spec_n1,
            _spec_full((256, 256)), _spec_full((1, 256)), _spec_n1,
        ],
        out_specs=[_split3(128), _split3(128)],
        out_shape=[jax.ShapeDtypeStruct((2, NP, 128), jnp.float32)] * 2,
    )(g, y, ddst, Wa, ba, dsrc)


def _b_body(g_ref, y_ref, dd_ref, w_ref, b_ref, ds_ref, yb_ref):
    dd = dd_ref[...]
    x0 = dd * (g_ref[0] + y_ref[0])
    x1 = dd * (g_ref[1] + y_ref[1])
    x = _softplus(_cat(x0, x1))
    yb = (_dot(x, w_ref[...]) + b_ref[...]) * ds_ref[...]
    _split_write(yb_ref, yb, 128)


def _tc_b(g, y, ddst, Wb, bb, dsrc):
    return pl.pallas_call(
        _b_body,
        grid=(GRID,),
        in_specs=[
            _split3(128), _split3(128), _spec_n1,
            _spec_full((256, 256)), _spec_full((1, 256)), _spec_n1,
        ],
        out_specs=_split3(128),
        out_shape=jax.ShapeDtypeStruct((2, NP, 128), jnp.float32),
    )(g, y, ddst, Wb, bb, dsrc)


def _c_mid_body(g_ref, y_ref, z_ref, dd_ref, w_ref, b_ref, ds_ref, dtc_ref,
                k_ref, ya_ref):
    dd = dd_ref[...]
    dtc = dtc_ref[0, 0]
    k0 = dd * (g_ref[0] + y_ref[0])
    k1 = dd * (g_ref[1] + y_ref[1])
    k_ref[0] = k0
    k_ref[1] = k1
    u = _cat(z_ref[0] + dtc * k0, z_ref[1] + dtc * k1)
    ya = (_dot(u, w_ref[...]) + b_ref[...]) * ds_ref[...]
    _split_write(ya_ref, ya, 128)


def _tc_c_mid(g, y, z, ddst, Wa, ba, dsrc, dtc):
    return pl.pallas_call(
        _c_mid_body,
        grid=(GRID,),
        in_specs=[
            _split3(128), _split3(128), _split3(128), _spec_n1,
            _spec_full((256, 256)), _spec_full((1, 256)), _spec_n1,
            _spec_full((1, 1)),
        ],
        out_specs=[_split3(128), _split3(128)],
        out_shape=[jax.ShapeDtypeStruct((2, NP, 128), jnp.float32)] * 2,
    )(g, y, z, ddst, Wa, ba, dsrc, dtc)


def _c_last_body(g_ref, y_ref, z_ref, k1_ref, k2_ref, k3_ref, dd_ref,
                 w_ref, b_ref, ds_ref, dt6_ref, zn_ref, ya_ref, *, split_out):
    dd = dd_ref[...]
    dt6 = dt6_ref[0, 0]
    zn = []
    for hh in range(2):
        k4 = dd * (g_ref[hh] + y_ref[hh])
        znh = z_ref[hh] + dt6 * (k1_ref[hh] + 2.0 * k2_ref[hh]
                                 + 2.0 * k3_ref[hh] + k4)
        zn_ref[hh] = znh
        zn.append(znh)
    ya = (_dot(_cat(zn[0], zn[1]), w_ref[...]) + b_ref[...]) * ds_ref[...]
    if split_out:
        _split_write(ya_ref, ya, 128)
    else:
        ya_ref[...] = ya


def _tc_c_last(g, y, z, k1, k2, k3, ddst, W, b, dsrc, dt6, split_out):
    wout = W.shape[1]
    if split_out:
        ya_spec = _split3(128)
        ya_shape = jax.ShapeDtypeStruct((2, NP, 128), jnp.float32)
    else:
        ya_spec = pl.BlockSpec((RB, wout), lambda i: (i, 0))
        ya_shape = jax.ShapeDtypeStruct((NP, wout), jnp.float32)
    return pl.pallas_call(
        functools.partial(_c_last_body, split_out=split_out),
        grid=(GRID,),
        in_specs=[
            _split3(128), _split3(128), _split3(128), _split3(128),
            _split3(128), _split3(128), _spec_n1,
            _spec_full((256, wout)), _spec_full((1, wout)), _spec_n1,
            _spec_full((1, 1)),
        ],
        out_specs=[_split3(128), ya_spec],
        out_shape=[jax.ShapeDtypeStruct((2, NP, 128), jnp.float32), ya_shape],
    )(g, y, z, k1, k2, k3, ddst, W, b, dsrc, dt6)


def _out_body(g_ref, y_ref, dd_ref, o_ref):
    o_ref[...] = dd_ref[...] * (g_ref[0] + g_ref[1] + y_ref[...])


def _tc_out(g, y, ddst):
    return pl.pallas_call(
        _out_body,
        grid=(GRID,),
        in_specs=[_split3(128), pl.BlockSpec((RB, 128), lambda i: (i, 0)),
                  _spec_n1],
        out_specs=pl.BlockSpec((RB, 128), lambda i: (i, 0)),
        out_shape=jax.ShapeDtypeStruct((NP, 128), jnp.float32),
    )(g, y, ddst)


def kernel(h, edge_index, t_span, W1, b1, Wa, ba, Wb, bb, W2, b2):
    f32 = jnp.float32
    src = edge_index[0]
    dst = edge_index[1]
    pad = jnp.full((EP - E,), PAD_NODE, jnp.int32)
    src_p = jnp.concatenate([src, pad])
    dst_p = jnp.concatenate([dst, pad])
    src2 = jnp.concatenate([src_p, src_p + NP]).reshape(2 * NBLK, EBLK)
    dst2 = dst_p.reshape(NBLK, EBLK)
    hist_idx = jnp.concatenate([src_p, dst_p])

    ones128 = jnp.ones((EBLK, 128), f32)
    zeros128 = jnp.zeros((EBLK, 128), f32)

    cnt = _hist(hist_idx, ones128, zeros128)
    dsrc, ddst = _tc_norms(cnt)

    h_pad = jnp.concatenate([h, jnp.zeros((NP - N, h.shape[1]), f32)])
    W1r, bar, bbr = W1, ba.reshape(1, -1), bb.reshape(1, -1)
    b1r, b2r = b1.reshape(1, -1), b2.reshape(1, -1)

    def spmm128(y3):
        g = _spmm128(y3.reshape(2 * NP, 128), src2, dst2, zeros128)
        return g.reshape(2, NP, 128)

    dts = t_span[1:] - t_span[:-1]

    y1 = _tc_in(h_pad, W1r, b1r, dsrc)
    g1 = spmm128(y1)
    z, ya = _tc_first(g1, y1, ddst, Wa, bar, dsrc)

    n_steps = t_span.shape[0] - 1
    yo = None
    for i in range(n_steps):
        dt = dts[i]
        ks = []
        for s_idx in range(4):
            ga = spmm128(ya)
            yb = _tc_b(ga, ya, ddst, Wb, bbr, dsrc)
            gb = spmm128(yb)
            if s_idx < 3:
                coeff = 0.5 if s_idx < 2 else 1.0
                dtc = (coeff * dt).reshape(1, 1)
                k, ya = _tc_c_mid(gb, yb, z, ddst, Wa, bar, dsrc, dtc)
                ks.append(k)
            else:
                dt6 = (dt / 6.0).reshape(1, 1)
                if i < n_steps - 1:
                    z, ya = _tc_c_last(gb, yb, z, ks[0], ks[1], ks[2],
                                       ddst, Wa, bar, dsrc, dt6, True)
                else:
                    z, yo = _tc_c_last(gb, yb, z, ks[0], ks[1], ks[2],
                                       ddst, W2, b2r, dsrc, dt6, False)

    go = _spmm_es(yo, src_p, dst_p, zeros128)
    go = go.reshape(2, NP, 128)
    out = _tc_out(go, yo, ddst)
    return out[:N]
